# baseline (speedup 1.0000x reference)
import functools

import jax
import jax.numpy as jnp
from jax import lax
from jax.experimental import pallas as pl
from jax.experimental.pallas import tpu as pltpu
from jax.experimental.pallas import tpu_sc as plsc

B = 4096
L = 50
BL = B * L
D_EMBED = 64
N_FEATURE = 16
D_FEATURE = 32
D_OUT = D_EMBED + D_FEATURE
TAB_W = 128
N_TOKEN_HALF = 500000

NC, NS = 2, 16
NW = NC * NS
TPW = BL // NW
CHUNK = 128
NCHUNK = TPW // CHUNK
GRP = CHUNK // 16


def _make_sc_kernel():
    mesh = plsc.VectorSubcoreMesh(core_axis_name="c", subcore_axis_name="s")

    @functools.partial(
        pl.kernel,
        mesh=mesh,
        out_type=jax.ShapeDtypeStruct((BL, D_OUT), jnp.float32),
        scratch_types=[
            pltpu.VMEM((TPW,), jnp.int32),
            pltpu.VMEM((CHUNK,), jnp.int32),
            pltpu.VMEM((CHUNK, D_OUT), jnp.float32),
            pltpu.VMEM((CHUNK, TAB_W), jnp.float32),
            pltpu.VMEM((CHUNK, N_FEATURE), jnp.float32),
            pltpu.VMEM((N_FEATURE, D_FEATURE), jnp.float32),
            pltpu.VMEM((D_FEATURE,), jnp.float32),
            pltpu.SemaphoreType.DMA,
        ],
    )
    def k(tok_hbm, feat_hbm, table_hbm, wt_hbm, b_hbm, out_hbm,
          idx_v, half_v, row_v, emb_v, feat_v, wt_v, b_v, sem):
        cid = lax.axis_index("c")
        sid = lax.axis_index("s")
        wid = cid * NS + sid

        tok_off = pl.multiple_of(wid * TPW, 8)
        pltpu.sync_copy(tok_hbm.at[pl.ds(tok_off, TPW)], idx_v)
        pltpu.sync_copy(wt_hbm, wt_v)
        pltpu.sync_copy(b_hbm, b_v)

        w_lo = [wt_v[kf, pl.ds(0, 16)] for kf in range(N_FEATURE)]
        w_hi = [wt_v[kf, pl.ds(16, 16)] for kf in range(N_FEATURE)]
        b_lo = b_v[pl.ds(0, 16)]
        b_hi = b_v[pl.ds(16, 16)]

        def chunk_body(j, carry):
            gbase = pl.multiple_of(wid * TPW + j * CHUNK, CHUNK)
            cbase = j * CHUNK
            for g in range(GRP):
                iv = idx_v[pl.ds(cbase + g * 16, 16)]
                half_v[pl.ds(g * 16, 16)] = iv >> 1
            pltpu.async_copy(table_hbm.at[half_v], emb_v, sem).wait()
            pltpu.sync_copy(feat_hbm.at[pl.ds(gbase, CHUNK)], feat_v)

            def tok_body(t, c2):
                f_row = feat_v[t, pl.ds(0, N_FEATURE)]
                acc0 = b_lo
                acc1 = b_hi
                for kf in range(N_FEATURE):
                    fs = f_row[kf]
                    acc0 = acc0 + fs * w_lo[kf]
                    acc1 = acc1 + fs * w_hi[kf]
                row_v[t, pl.ds(D_EMBED, 16)] = acc0
                row_v[t, pl.ds(D_EMBED + 16, 16)] = acc1
                return c2

            lax.fori_loop(0, CHUNK, tok_body, 0)

            for g in range(GRP):
                offv = (idx_v[pl.ds(cbase + g * 16, 16)] & 1) * D_EMBED
                for i in range(16):
                    t = g * 16 + i
                    off_s = offv[i]
                    for c in range(0, D_EMBED, 16):
                        row_v[t, pl.ds(c, 16)] = emb_v[t, pl.ds(off_s + c, 16)]

            pltpu.sync_copy(row_v, out_hbm.at[pl.ds(gbase, CHUNK)])
            return carry

        lax.fori_loop(0, NCHUNK, chunk_body, 0)

    return k


_sc_kernel = _make_sc_kernel()


def kernel(tokens, features, embed_table, proj_W, proj_b):
    tok = tokens.reshape(BL).astype(jnp.int32)
    feat = features.reshape(BL, N_FEATURE)
    table2 = embed_table.reshape(N_TOKEN_HALF, TAB_W)
    wt = proj_W.T
    out = _sc_kernel(tok, feat, table2, wt, proj_b)
    return out.reshape(B, L, D_OUT)


# speedup vs baseline: 1.0115x; 1.0115x over previous
import functools

import jax
import jax.numpy as jnp
from jax import lax
from jax.experimental import pallas as pl
from jax.experimental.pallas import tpu as pltpu
from jax.experimental.pallas import tpu_sc as plsc

B = 4096
L = 50
D_EMBED = 64
N_FEATURE = 16
D_FEATURE = 32
D_OUT = D_EMBED + D_FEATURE
TAB_W = 128
N_TOKEN_HALF = 500000

NC, NS = 2, 16
NW = NC * NS
BPW = B // NW


def _make_sc_kernel():
    mesh = plsc.VectorSubcoreMesh(core_axis_name="c", subcore_axis_name="s")

    @functools.partial(
        pl.kernel,
        mesh=mesh,
        compiler_params=pltpu.CompilerParams(needs_layout_passes=False),
        out_type=jax.ShapeDtypeStruct((L, D_OUT, B), jnp.float32),
        scratch_types=[
            pltpu.VMEM((L, BPW), jnp.int32),
            pltpu.VMEM((BPW,), jnp.int32),
            pltpu.VMEM((BPW,), jnp.int32),
            pltpu.VMEM((BPW, TAB_W), jnp.float32),
            pltpu.VMEM((N_FEATURE, BPW), jnp.float32),
            pltpu.VMEM((D_OUT, BPW), jnp.float32),
            pltpu.VMEM((24, 128), jnp.float32),
            pltpu.SMEM((N_FEATURE * D_FEATURE,), jnp.float32),
            pltpu.SMEM((D_FEATURE,), jnp.float32),
            pltpu.SemaphoreType.DMA,
        ],
    )
    def k(tok_hbm, feat_hbm, table_hbm, aux_hbm, out_hbm,
          tok_v, half_v, par_v, emb_v, feat_v, tile_v, aux_v,
          w_sm, b_sm, sem):
        cid = lax.axis_index("c")
        sid = lax.axis_index("s")
        wid = cid * NS + sid
        boff = pl.multiple_of(wid * BPW, BPW)

        pltpu.sync_copy(tok_hbm.at[:, pl.ds(boff, BPW)], tok_v)
        pltpu.sync_copy(aux_hbm, aux_v)
        for kf in range(N_FEATURE):
            for jh in range(D_FEATURE // 16):
                wv = aux_v[kf, pl.ds(jh * 16, 16)]
                for i in range(16):
                    w_sm[kf * D_FEATURE + jh * 16 + i] = wv[i]
        for jh in range(D_FEATURE // 16):
            bv = aux_v[N_FEATURE, pl.ds(jh * 16, 16)]
            for i in range(16):
                b_sm[jh * 16 + i] = bv[i]

        lane = lax.iota(jnp.int32, 16)

        def l_body(l, carry):
            for g in range(BPW // 16):
                t16 = tok_v[l, pl.ds(g * 16, 16)]
                half_v[pl.ds(g * 16, 16)] = t16 >> 1
                par_v[pl.ds(g * 16, 16)] = (t16 & 1) * D_EMBED
            pltpu.async_copy(table_hbm.at[half_v], emb_v, sem).wait()
            pltpu.sync_copy(feat_hbm.at[l, :, pl.ds(boff, BPW)], feat_v)

            for g in range(BPW // 16):
                fv = [feat_v[kf, pl.ds(g * 16, 16)] for kf in range(N_FEATURE)]

                def j_body(j, c2):
                    acc = fv[0] * w_sm[j] + b_sm[j]
                    for kf in range(1, N_FEATURE):
                        acc = acc + fv[kf] * w_sm[kf * D_FEATURE + j]
                    tile_v[D_EMBED + j, pl.ds(g * 16, 16)] = acc
                    return c2

                lax.fori_loop(0, D_FEATURE, j_body, 0)

            for g in range(BPW // 16):
                rows = lane + (g * 16)
                cols0 = par_v[pl.ds(g * 16, 16)]

                def c_body(c8, c2):
                    for dc in range(8):
                        c = c8 * 8 + dc
                        val = plsc.load_gather(emb_v, [rows, cols0 + c])
                        tile_v[c, pl.ds(g * 16, 16)] = val
                    return c2

                lax.fori_loop(0, D_EMBED // 8, c_body, 0)

            pltpu.sync_copy(tile_v, out_hbm.at[l, :, pl.ds(boff, BPW)])
            return carry

        lax.fori_loop(0, L, l_body, 0)

    return k


_sc_kernel = _make_sc_kernel()


def kernel(tokens, features, embed_table, proj_W, proj_b):
    tok_t = tokens.astype(jnp.int32).T
    feat_t = features.transpose(1, 2, 0)
    table2 = embed_table.reshape(N_TOKEN_HALF, TAB_W)
    aux = jnp.zeros((24, 128), jnp.float32)
    aux = aux.at[:N_FEATURE, :D_FEATURE].set(proj_W.T)
    aux = aux.at[N_FEATURE, :D_FEATURE].set(proj_b)
    out = _sc_kernel(tok_t, feat_t, table2, aux)
    return out.transpose(2, 0, 1)
